# baseline (speedup 1.0000x reference)
import functools

import jax
import jax.numpy as jnp
from jax import lax
from jax.experimental import pallas as pl
from jax.experimental.pallas import tpu as pltpu
from jax.experimental.pallas import tpu_sc as plsc

B, S, H = 2, 2048, 1024
E = 8
K = 2
F = 2048
N = B * S
NK = N * K
T = 512
P = NK + E * T
NT = P // T
TS = 512
NW = 32


def _mlp_tile_kernel(tile_e_ref, tile_valid_ref, xs_ref, wg_ref, wu_ref,
                     wd_ref, ys_ref):
    i = pl.program_id(0)

    @pl.when(tile_valid_ref[i] == 1)
    def _():
        xs = xs_ref[...].astype(jnp.bfloat16)
        wg = wg_ref[0].astype(jnp.bfloat16)
        wu = wu_ref[0].astype(jnp.bfloat16)
        wd = wd_ref[0].astype(jnp.bfloat16)
        g = jnp.dot(xs, wg, preferred_element_type=jnp.float32)
        u = jnp.dot(xs, wu, preferred_element_type=jnp.float32)
        h = (g * jax.nn.sigmoid(g) * u).astype(jnp.bfloat16)
        ys_ref[...] = jnp.dot(h, wd, preferred_element_type=jnp.float32)


def _grouped_mlp(xs, wg, wu, wd, tile_e, tile_valid, tile_rows):
    n_tiles = xs.shape[0] // tile_rows
    grid_spec = pltpu.PrefetchScalarGridSpec(
        num_scalar_prefetch=2,
        grid=(n_tiles,),
        in_specs=[
            pl.BlockSpec((tile_rows, H), lambda i, te, tv: (i, 0)),
            pl.BlockSpec((1, H, F), lambda i, te, tv: (te[i], 0, 0)),
            pl.BlockSpec((1, H, F), lambda i, te, tv: (te[i], 0, 0)),
            pl.BlockSpec((1, F, H), lambda i, te, tv: (te[i], 0, 0)),
        ],
        out_specs=pl.BlockSpec((tile_rows, H), lambda i, te, tv: (i, 0)),
    )
    return pl.pallas_call(
        _mlp_tile_kernel,
        grid_spec=grid_spec,
        out_shape=jax.ShapeDtypeStruct((xs.shape[0], H), jnp.float32),
        compiler_params=pltpu.CompilerParams(
            vmem_limit_bytes=100 * 1024 * 1024),
    )(tile_e, tile_valid, xs, wg, wu, wd)


_SC_MESH = plsc.VectorSubcoreMesh(core_axis_name="c", subcore_axis_name="s")

_D_CH = 64
_D_PER_W = NK // NW


def _sc_dispatch(x, pos_flat, tok_flat):
    @functools.partial(
        pl.kernel,
        mesh=_SC_MESH,
        out_type=jax.ShapeDtypeStruct((P, H), jnp.float32),
        scratch_types=[
            pltpu.VMEM((2, _D_CH), jnp.int32),
            pltpu.VMEM((_D_CH, H), jnp.float32),
            pltpu.SemaphoreType.DMA,
            pltpu.SemaphoreType.DMA,
        ],
    )
    def body(x_hbm, pos_hbm, tok_hbm, xs_hbm, idx_v, rows_v, sem_g, sem_s):
        wid = lax.axis_index("s") * 2 + lax.axis_index("c")
        for j in range(_D_PER_W // _D_CH):
            base = wid * _D_PER_W + j * _D_CH
            pltpu.sync_copy(tok_hbm.at[pl.ds(base, _D_CH)], idx_v.at[0])
            pltpu.sync_copy(pos_hbm.at[pl.ds(base, _D_CH)], idx_v.at[1])
            pltpu.async_copy(x_hbm.at[idx_v.at[0]], rows_v, sem_g).wait()
            pltpu.async_copy(rows_v, xs_hbm.at[idx_v.at[1]], sem_s).wait()

    return body(x, pos_flat, tok_flat)


_C_CH = 32
_C_PER_W = N // NW


def _sc_combine(ys, shared, p0, p1, w0, w1):
    @functools.partial(
        pl.kernel,
        mesh=_SC_MESH,
        out_type=jax.ShapeDtypeStruct((N, H), jnp.float32),
        scratch_types=[
            pltpu.VMEM((2, _C_CH), jnp.int32),
            pltpu.VMEM((_C_CH, 16), jnp.float32),
            pltpu.VMEM((_C_CH, 16), jnp.float32),
            pltpu.VMEM((_C_CH, H), jnp.float32),
            pltpu.VMEM((_C_CH, H), jnp.float32),
            pltpu.VMEM((_C_CH, H), jnp.float32),
            pltpu.SemaphoreType.DMA,
            pltpu.SemaphoreType.DMA,
            pltpu.SemaphoreType.DMA,
        ],
    )
    def body(ys_hbm, sh_hbm, p0_hbm, p1_hbm, w0_hbm, w1_hbm, out_hbm,
             pidx_v, w0_v, w1_v, r0_v, r1_v, sh_v, sem0, sem1, sem2):
        wid = lax.axis_index("s") * 2 + lax.axis_index("c")
        for j in range(_C_PER_W // _C_CH):
            base = wid * _C_PER_W + j * _C_CH
            pltpu.sync_copy(p0_hbm.at[pl.ds(base, _C_CH)], pidx_v.at[0])
            pltpu.sync_copy(p1_hbm.at[pl.ds(base, _C_CH)], pidx_v.at[1])
            pltpu.sync_copy(w0_hbm.at[pl.ds(base, _C_CH)], w0_v)
            pltpu.sync_copy(w1_hbm.at[pl.ds(base, _C_CH)], w1_v)
            c0 = pltpu.async_copy(ys_hbm.at[pidx_v.at[0]], r0_v, sem0)
            c1 = pltpu.async_copy(ys_hbm.at[pidx_v.at[1]], r1_v, sem1)
            c2 = pltpu.async_copy(sh_hbm.at[pl.ds(base, _C_CH)], sh_v, sem2)
            c0.wait()
            c1.wait()
            c2.wait()

            def row_body(r, _):
                w0s = w0_v[r, pl.ds(0, 16)]
                w1s = w1_v[r, pl.ds(0, 16)]

                def col_body(c, _):
                    off = c * 128
                    for u in range(8):
                        o = off + u * 16
                        v = (r0_v[r, pl.ds(o, 16)] * w0s
                             + r1_v[r, pl.ds(o, 16)] * w1s
                             + sh_v[r, pl.ds(o, 16)])
                        r0_v[r, pl.ds(o, 16)] = v
                    return 0

                lax.fori_loop(0, H // 128, col_body, 0)
                return 0

            lax.fori_loop(0, _C_CH, row_body, 0)
            pltpu.sync_copy(r0_v, out_hbm.at[pl.ds(base, _C_CH)])

    return body(ys, shared, p0, p1, w0, w1)


def kernel(hidden_states, Wg, We_gate, We_up, We_down, Ws_gate, Ws_up,
           Ws_down):
    x = hidden_states.reshape(N, H)

    logits = x @ Wg
    probs = jax.nn.softmax(logits, axis=-1)
    topk_w, topk_i = jax.lax.top_k(probs, K)
    topk_w = topk_w / jnp.sum(topk_w, axis=-1, keepdims=True)

    flat_e = topk_i.reshape(-1).astype(jnp.int32)
    oh = (flat_e[:, None] == jnp.arange(E, dtype=jnp.int32)[None, :]).astype(jnp.int32)
    ranks_all = jnp.cumsum(oh, axis=0) - oh
    rank = jnp.take_along_axis(ranks_all, flat_e[:, None], axis=1)[:, 0]
    counts = jnp.sum(oh, axis=0)
    padded = ((counts + T - 1) // T) * T
    ends = jnp.cumsum(padded)
    offs = ends - padded
    pos = offs[flat_e] + rank

    tile_start = jnp.arange(NT, dtype=jnp.int32) * T
    tile_e = jnp.minimum(
        jnp.sum((tile_start[:, None] >= ends[None, :]).astype(jnp.int32), axis=1),
        E - 1).astype(jnp.int32)
    tile_valid = (tile_start < ends[E - 1]).astype(jnp.int32)

    pos2 = pos.reshape(N, K)
    p0 = pos2[:, 0]
    p1 = pos2[:, 1]
    ar = jnp.arange(N, dtype=jnp.int32)
    pos_flat = jnp.concatenate([p0, p1])
    tok_flat = jnp.concatenate([ar, ar])
    xs = _sc_dispatch(x, pos_flat, tok_flat)

    ys = _grouped_mlp(xs, We_gate, We_up, We_down, tile_e, tile_valid, T)

    shared = _grouped_mlp(
        x, Ws_gate[None], Ws_up[None], Ws_down[None],
        jnp.zeros((N // TS,), jnp.int32), jnp.ones((N // TS,), jnp.int32), TS)

    w0b = jnp.broadcast_to(topk_w[:, 0:1], (N, 16))
    w1b = jnp.broadcast_to(topk_w[:, 1:2], (N, 16))
    out = _sc_combine(ys, shared, p0, p1, w0b, w1b)
    return out.reshape(B, S, H)


# speedup vs baseline: 1.0377x; 1.0377x over previous
import functools

import jax
import jax.numpy as jnp
from jax import lax
from jax.experimental import pallas as pl
from jax.experimental.pallas import tpu as pltpu
from jax.experimental.pallas import tpu_sc as plsc

B, S, H = 2, 2048, 1024
E = 8
K = 2
F = 2048
N = B * S
NK = N * K
T = 512
P = NK + E * T
NT = P // T
TS = 512
NW = 32


def _mlp_tile_kernel(tile_e_ref, tile_valid_ref, xs_ref, wg_ref, wu_ref,
                     wd_ref, ys_ref):
    i = pl.program_id(0)

    @pl.when(tile_valid_ref[i] == 1)
    def _():
        xs = xs_ref[...].astype(jnp.bfloat16)
        wg = wg_ref[0].astype(jnp.bfloat16)
        wu = wu_ref[0].astype(jnp.bfloat16)
        wd = wd_ref[0].astype(jnp.bfloat16)
        g = jnp.dot(xs, wg, preferred_element_type=jnp.float32)
        u = jnp.dot(xs, wu, preferred_element_type=jnp.float32)
        h = (g * jax.nn.sigmoid(g) * u).astype(jnp.bfloat16)
        ys_ref[...] = jnp.dot(h, wd, preferred_element_type=jnp.float32)


def _grouped_mlp(xs, wg, wu, wd, tile_e, tile_valid, tile_rows):
    n_tiles = xs.shape[0] // tile_rows
    grid_spec = pltpu.PrefetchScalarGridSpec(
        num_scalar_prefetch=2,
        grid=(n_tiles,),
        in_specs=[
            pl.BlockSpec((tile_rows, H), lambda i, te, tv: (i, 0)),
            pl.BlockSpec((1, H, F), lambda i, te, tv: (te[i], 0, 0)),
            pl.BlockSpec((1, H, F), lambda i, te, tv: (te[i], 0, 0)),
            pl.BlockSpec((1, F, H), lambda i, te, tv: (te[i], 0, 0)),
        ],
        out_specs=pl.BlockSpec((tile_rows, H), lambda i, te, tv: (i, 0)),
    )
    return pl.pallas_call(
        _mlp_tile_kernel,
        grid_spec=grid_spec,
        out_shape=jax.ShapeDtypeStruct((xs.shape[0], H), jnp.float32),
        compiler_params=pltpu.CompilerParams(
            vmem_limit_bytes=100 * 1024 * 1024),
    )(tile_e, tile_valid, xs, wg, wu, wd)


def _sc_mesh():
    return plsc.VectorSubcoreMesh(core_axis_name="c", subcore_axis_name="s")

_D_CH = 64
_D_PER_W = NK // NW


def _sc_dispatch(x, pos_flat, tok_flat):
    @functools.partial(
        pl.kernel,
        mesh=_sc_mesh(),
        out_type=jax.ShapeDtypeStruct((P, H), jnp.float32),
        scratch_types=[
            pltpu.VMEM((2, _D_CH), jnp.int32),
            pltpu.VMEM((_D_CH, H), jnp.float32),
            pltpu.SemaphoreType.DMA,
            pltpu.SemaphoreType.DMA,
        ],
    )
    def body(x_hbm, pos_hbm, tok_hbm, xs_hbm, idx_v, rows_v, sem_g, sem_s):
        wid = lax.axis_index("s") * 2 + lax.axis_index("c")
        for j in range(_D_PER_W // _D_CH):
            base = wid * _D_PER_W + j * _D_CH
            pltpu.sync_copy(tok_hbm.at[pl.ds(base, _D_CH)], idx_v.at[0])
            pltpu.sync_copy(pos_hbm.at[pl.ds(base, _D_CH)], idx_v.at[1])
            pltpu.async_copy(x_hbm.at[idx_v.at[0]], rows_v, sem_g).wait()
            pltpu.async_copy(rows_v, xs_hbm.at[idx_v.at[1]], sem_s).wait()

    return body(x, pos_flat, tok_flat)


_C_CH = 32
_C_PER_W = N // NW


def _sc_combine(ys, shared, p0, p1, w0, w1):
    @functools.partial(
        pl.kernel,
        mesh=_sc_mesh(),
        out_type=jax.ShapeDtypeStruct((N, H), jnp.float32),
        scratch_types=[
            pltpu.VMEM((2, _C_CH), jnp.int32),
            pltpu.VMEM((_C_CH, 16), jnp.float32),
            pltpu.VMEM((_C_CH, 16), jnp.float32),
            pltpu.VMEM((_C_CH, H), jnp.float32),
            pltpu.VMEM((_C_CH, H), jnp.float32),
            pltpu.VMEM((_C_CH, H), jnp.float32),
            pltpu.SemaphoreType.DMA,
            pltpu.SemaphoreType.DMA,
            pltpu.SemaphoreType.DMA,
        ],
    )
    def body(ys_hbm, sh_hbm, p0_hbm, p1_hbm, w0_hbm, w1_hbm, out_hbm,
             pidx_v, w0_v, w1_v, r0_v, r1_v, sh_v, sem0, sem1, sem2):
        wid = lax.axis_index("s") * 2 + lax.axis_index("c")
        for j in range(_C_PER_W // _C_CH):
            base = wid * _C_PER_W + j * _C_CH
            pltpu.sync_copy(p0_hbm.at[pl.ds(base, _C_CH)], pidx_v.at[0])
            pltpu.sync_copy(p1_hbm.at[pl.ds(base, _C_CH)], pidx_v.at[1])
            pltpu.sync_copy(w0_hbm.at[pl.ds(base, _C_CH)], w0_v)
            pltpu.sync_copy(w1_hbm.at[pl.ds(base, _C_CH)], w1_v)
            c0 = pltpu.async_copy(ys_hbm.at[pidx_v.at[0]], r0_v, sem0)
            c1 = pltpu.async_copy(ys_hbm.at[pidx_v.at[1]], r1_v, sem1)
            c2 = pltpu.async_copy(sh_hbm.at[pl.ds(base, _C_CH)], sh_v, sem2)
            c0.wait()
            c1.wait()
            c2.wait()

            def row_body(r, _):
                w0s = w0_v[r, pl.ds(0, 16)]
                w1s = w1_v[r, pl.ds(0, 16)]

                def col_body(c, _):
                    off = c * 16
                    v = (r0_v[r, pl.ds(off, 16)] * w0s
                         + r1_v[r, pl.ds(off, 16)] * w1s
                         + sh_v[r, pl.ds(off, 16)])
                    r0_v[r, pl.ds(off, 16)] = v
                    return 0

                lax.fori_loop(0, H // 16, col_body, 0)
                return 0

            lax.fori_loop(0, _C_CH, row_body, 0)
            pltpu.sync_copy(r0_v, out_hbm.at[pl.ds(base, _C_CH)])

    return body(ys, shared, p0, p1, w0, w1)


_GRP = 128
_NG = N // _GRP


def _gate_kernel(x_ref, wg_ref, route_ref, tiles_ref):
    x = x_ref[...]
    logits = jnp.dot(x, wg_ref[...], preferred_element_type=jnp.float32)

    m = jnp.max(logits, axis=1, keepdims=True)
    ex = jnp.exp(logits - m)
    probs = ex / jnp.sum(ex, axis=1, keepdims=True)

    iota8 = lax.broadcasted_iota(jnp.int32, (N, E), 1)
    m1 = jnp.max(probs, axis=1, keepdims=True)
    i1 = jnp.min(jnp.where(probs == m1, iota8, E), axis=1, keepdims=True)
    oh0 = (iota8 == i1)
    probs2 = jnp.where(oh0, -1.0, probs)
    m2 = jnp.max(probs2, axis=1, keepdims=True)
    i2 = jnp.min(jnp.where(probs2 == m2, iota8, E), axis=1, keepdims=True)
    oh1 = (iota8 == i2)
    wsum = m1 + m2
    w0 = m1 / wsum
    w1 = m2 / wsum

    ohc = (oh0 | oh1).astype(jnp.float32)
    ig = lax.broadcasted_iota(jnp.int32, (_NG, N), 0)
    it = lax.broadcasted_iota(jnp.int32, (_NG, N), 1)
    wseg = (it // _GRP == ig).astype(jnp.float32)
    igt = lax.broadcasted_iota(jnp.int32, (N, _NG), 1)
    itt = lax.broadcasted_iota(jnp.int32, (N, _NG), 0)
    wseg_t = (itt // _GRP == igt).astype(jnp.float32)
    ra = lax.broadcasted_iota(jnp.int32, (_NG, _NG), 0)
    ca = lax.broadcasted_iota(jnp.int32, (_NG, _NG), 1)
    sl_g = (ca < ra).astype(jnp.float32)
    rb = lax.broadcasted_iota(jnp.int32, (_GRP, _GRP), 0)
    cb = lax.broadcasted_iota(jnp.int32, (_GRP, _GRP), 1)
    sl_t = (cb < rb).astype(jnp.float32)

    hi = lax.Precision.HIGHEST
    gs = jnp.dot(wseg, ohc, precision=hi,
                 preferred_element_type=jnp.float32)
    pg = jnp.dot(sl_g, gs, precision=hi,
                 preferred_element_type=jnp.float32)
    pgb = jnp.dot(wseg_t, pg, precision=hi,
                  preferred_element_type=jnp.float32)
    intra = jnp.concatenate(
        [jnp.dot(sl_t, ohc[g * _GRP:(g + 1) * _GRP, :], precision=hi,
                 preferred_element_type=jnp.float32) for g in range(_NG)],
        axis=0)
    cum = pgb + intra

    rank0 = jnp.sum(cum * oh0, axis=1, keepdims=True)
    rank1 = jnp.sum(cum * oh1, axis=1, keepdims=True)

    counts = jnp.sum(ohc, axis=0, keepdims=True)
    padded = jnp.floor((counts + (T - 1)) * (1.0 / T)) * T
    re = lax.broadcasted_iota(jnp.int32, (E, E), 0)
    ce = lax.broadcasted_iota(jnp.int32, (E, E), 1)
    ut_incl = (re <= ce).astype(jnp.float32)
    ends = jnp.dot(padded, ut_incl, precision=hi,
                   preferred_element_type=jnp.float32)
    offs = ends - padded

    pos0 = jnp.sum(offs * oh0, axis=1, keepdims=True) + rank0
    pos1 = jnp.sum(offs * oh1, axis=1, keepdims=True) + rank1

    route_ref[...] = jnp.concatenate(
        [pos0, pos1, w0, w1, jnp.zeros((N, 4), jnp.float32)], axis=1)

    starts = (lax.broadcasted_iota(jnp.int32, (NT, E), 0) * T).astype(
        jnp.float32)
    endsb = jnp.broadcast_to(ends, (NT, E))
    tile_e = jnp.minimum(
        jnp.sum((starts >= endsb).astype(jnp.float32), axis=1, keepdims=True),
        float(E - 1))
    tile_valid = (starts[:, 0:1] < endsb[:, E - 1:E]).astype(jnp.float32)
    tiles_ref[...] = jnp.concatenate(
        [tile_e, tile_valid, jnp.zeros((NT, 6), jnp.float32)], axis=1)


def _gate_route(x, Wg):
    return pl.pallas_call(
        _gate_kernel,
        out_shape=[jax.ShapeDtypeStruct((N, E), jnp.float32),
                   jax.ShapeDtypeStruct((NT, E), jnp.float32)],
    )(x, Wg)


def kernel(hidden_states, Wg, We_gate, We_up, We_down, Ws_gate, Ws_up,
           Ws_down):
    x = hidden_states.reshape(N, H)

    route, tiles = _gate_route(x, Wg)
    p0 = route[:, 0].astype(jnp.int32)
    p1 = route[:, 1].astype(jnp.int32)
    w0 = route[:, 2]
    w1 = route[:, 3]
    tile_e = tiles[:, 0].astype(jnp.int32)
    tile_valid = tiles[:, 1].astype(jnp.int32)
    ar = jnp.arange(N, dtype=jnp.int32)
    pos_flat = jnp.concatenate([p0, p1])
    tok_flat = jnp.concatenate([ar, ar])
    xs = _sc_dispatch(x, pos_flat, tok_flat)

    ys = _grouped_mlp(xs, We_gate, We_up, We_down, tile_e, tile_valid, T)

    shared = _grouped_mlp(
        x, Ws_gate[None], Ws_up[None], Ws_down[None],
        jnp.zeros((N // TS,), jnp.int32), jnp.ones((N // TS,), jnp.int32), TS)

    w0b = jnp.broadcast_to(w0[:, None], (N, 16))
    w1b = jnp.broadcast_to(w1[:, None], (N, 16))
    out = _sc_combine(ys, shared, p0, p1, w0b, w1b)
    return out.reshape(B, S, H)
